# baseline (speedup 1.0000x reference)
import functools

import jax
import jax.numpy as jnp
from jax import lax
from jax.experimental import pallas as pl
from jax.experimental.pallas import tpu as pltpu
from jax.experimental.pallas import tpu_sc as plsc

_N = 10000
_E = 320000
_D = 128
_ND = 128

_BATCH = 128
_ROWS_PER_TILE = 100
_NTILES = 25


def _deg_body(edges_hbm, out_hbm, idx2_v, ones_v, zeros_v, hist_s, sem):
    c = lax.axis_index("c")
    s = lax.axis_index("s")
    wid = s * 2 + c

    widc = jnp.minimum(wid, _NTILES - 1)
    src = edges_hbm.at[:, pl.ds(widc * _ROWS_PER_TILE * _BATCH,
                                _ROWS_PER_TILE * _BATCH)]

    @pl.when(wid < _NTILES)
    def _():
        pltpu.async_copy(src, idx2_v, sem)

    zero16 = jnp.zeros((16,), jnp.int32)
    one16 = jnp.ones((16,), jnp.int32)

    def fill(j, carry):
        zeros_v[pl.ds(j * 16, 16)] = zero16
        return carry

    lax.fori_loop(0, 40, fill, 0)

    def fill1(j, carry):
        ones_v[pl.ds(j * 16, 16)] = one16
        return carry

    lax.fori_loop(0, 8, fill1, 0)

    @pl.when(s < 15)
    def _():
        pltpu.sync_copy(zeros_v, hist_s.at[pl.ds(s * 640, 640)])

    @pl.when(s == 15)
    def _():
        pltpu.sync_copy(zeros_v.at[pl.ds(0, 400)], hist_s.at[pl.ds(9600, 400)])

    @pl.when(wid < _NTILES)
    def _():
        pltpu.make_async_copy(src, idx2_v, sem).wait()

    plsc.subcore_barrier()

    @pl.when(wid < _NTILES)
    def _():
        def chunk(cj, carry):
            base = cj * 5
            descs = [
                pltpu.async_copy(
                    ones_v,
                    hist_s.at[idx2_v.at[1, pl.ds((base + u) * _BATCH,
                                                 _BATCH)]],
                    sem, add=True)
                for u in range(5)
            ]
            for d in descs:
                d.wait()
            return carry

        lax.fori_loop(0, _ROWS_PER_TILE // 5, chunk, 0)

    plsc.subcore_barrier()

    @pl.when(s == 0)
    def _():
        pltpu.sync_copy(hist_s, out_hbm.at[c, 0])


@functools.cache
def _deg_sc():
    return pl.kernel(
        _deg_body,
        mesh=plsc.VectorSubcoreMesh(core_axis_name="c", subcore_axis_name="s"),
        out_type=jax.ShapeDtypeStruct((2, 1, _N), jnp.int32),
        scratch_types=[
            pltpu.VMEM((2, _ROWS_PER_TILE * _BATCH), jnp.int32),
            pltpu.VMEM((_BATCH,), jnp.int32),
            pltpu.VMEM((640,), jnp.int32),
            pltpu.VMEM_SHARED((_N,), jnp.int32),
            pltpu.SemaphoreType.DMA,
        ],
    )


_BLK = 2000


def _quant_body(part_ref, gama_ref, bit_ref, fea_ref,
                out_ref, bs_ref, sq_scr):
    i = pl.program_id(0)

    @pl.when(i == 0)
    def _():
        p = part_ref[...]
        deg = p[0] + p[1]
        idx = jnp.minimum(deg, _ND - 1)
        kk = lax.broadcasted_iota(jnp.int32, (_ND, _N), 0)
        onehot_t = (kk == idx).astype(jnp.float32)

        g = jnp.maximum(jnp.abs(gama_ref[...]), 1e-12)
        b = jnp.clip(jnp.round(bit_ref[...]), 1.0, 32.0)
        qm = jnp.exp2(b - 1.0) - 1.0
        tab = jnp.concatenate([g, qm], axis=1)
        sq_scr[...] = lax.dot_general(
            onehot_t, tab,
            dimension_numbers=(((0,), (0,)), ((), ())),
            preferred_element_type=jnp.float32)

        occ = jnp.max(onehot_t, axis=1, keepdims=True)
        total = lax.dot_general(occ, bit_ref[...],
                                dimension_numbers=(((0,), (0,)), ((), ())),
                                preferred_element_type=jnp.float32)
        bs_ref[...] = total * (_D / 8.0 / 1024.0)

    sq = sq_scr[pl.ds(i * _BLK, _BLK), :]
    sc = sq[:, 0:1]
    qmax = sq[:, 1:2]
    q = jnp.clip(jnp.round(fea_ref[...] / sc), -qmax, qmax)
    out_ref[...] = q * sc


def _quant_tc(partials, gama, bit, fea):
    return pl.pallas_call(
        _quant_body,
        grid=(_N // _BLK,),
        in_specs=[
            pl.BlockSpec((2, 1, _N), lambda i: (0, 0, 0)),
            pl.BlockSpec((_ND, 1), lambda i: (0, 0)),
            pl.BlockSpec((_ND, 1), lambda i: (0, 0)),
            pl.BlockSpec((_BLK, _D), lambda i: (i, 0)),
        ],
        out_specs=[
            pl.BlockSpec((_BLK, _D), lambda i: (i, 0)),
            pl.BlockSpec((1, 1), lambda i: (0, 0)),
        ],
        out_shape=[
            jax.ShapeDtypeStruct((_N, _D), jnp.float32),
            jax.ShapeDtypeStruct((1, 1), jnp.float32),
        ],
        scratch_shapes=[pltpu.VMEM((_N, 2), jnp.float32)],
    )(partials, gama, bit, fea)


def kernel(fea, edge_index, gama, bit):
    partials = _deg_sc()(edge_index)
    fea_q, bs = _quant_tc(partials, gama, bit, fea)
    return fea_q, bs[0, 0]


# speedup vs baseline: 1.0333x; 1.0333x over previous
import functools

import jax
import jax.numpy as jnp
from jax import lax
from jax.experimental import pallas as pl
from jax.experimental.pallas import tpu as pltpu
from jax.experimental.pallas import tpu_sc as plsc

_N = 10000
_E = 320000
_D = 128
_ND = 128

_BATCH = 128
_ROWS_PER_TILE = 100
_NTILES = 25


def _deg_body(edges_hbm, out_hbm, idx2_v, ones_v, zeros_v, hist_s, sem):
    c = lax.axis_index("c")
    s = lax.axis_index("s")
    wid = s * 2 + c

    widc = jnp.minimum(wid, _NTILES - 1)
    src = edges_hbm.at[:, pl.ds(widc * _ROWS_PER_TILE * _BATCH,
                                _ROWS_PER_TILE * _BATCH)]

    @pl.when(wid < _NTILES)
    def _():
        pltpu.async_copy(src, idx2_v, sem)

    zero16 = jnp.zeros((16,), jnp.int32)
    one16 = jnp.ones((16,), jnp.int32)

    def fill(j, carry):
        zeros_v[pl.ds(j * 16, 16)] = zero16
        return carry

    lax.fori_loop(0, 40, fill, 0)

    def fill1(j, carry):
        ones_v[pl.ds(j * 16, 16)] = one16
        return carry

    lax.fori_loop(0, 8, fill1, 0)

    @pl.when(s < 15)
    def _():
        pltpu.sync_copy(zeros_v, hist_s.at[pl.ds(s * 640, 640)])

    @pl.when(s == 15)
    def _():
        pltpu.sync_copy(zeros_v.at[pl.ds(0, 400)], hist_s.at[pl.ds(9600, 400)])

    @pl.when(wid < _NTILES)
    def _():
        pltpu.make_async_copy(src, idx2_v, sem).wait()

    plsc.subcore_barrier()

    @pl.when(wid < _NTILES)
    def _():
        def fire(base):
            for u in range(5):
                pltpu.async_copy(
                    ones_v,
                    hist_s.at[idx2_v.at[1, pl.ds((base + u) * _BATCH,
                                                 _BATCH)]],
                    sem, add=True)

        def drain5():
            for _u in range(5):
                pltpu.make_async_copy(
                    ones_v, hist_s.at[idx2_v.at[1, pl.ds(0, _BATCH)]],
                    sem).wait()

        fire(0)

        def chunk(cj, carry):
            fire(cj * 5)
            drain5()
            return carry

        lax.fori_loop(1, _ROWS_PER_TILE // 5, chunk, 0)
        drain5()

    plsc.subcore_barrier()

    @pl.when(s == 0)
    def _():
        pltpu.sync_copy(hist_s, out_hbm.at[c, 0])


@functools.cache
def _deg_sc():
    return pl.kernel(
        _deg_body,
        mesh=plsc.VectorSubcoreMesh(core_axis_name="c", subcore_axis_name="s"),
        out_type=jax.ShapeDtypeStruct((2, 1, _N), jnp.int32),
        scratch_types=[
            pltpu.VMEM((2, _ROWS_PER_TILE * _BATCH), jnp.int32),
            pltpu.VMEM((_BATCH,), jnp.int32),
            pltpu.VMEM((640,), jnp.int32),
            pltpu.VMEM_SHARED((_N,), jnp.int32),
            pltpu.SemaphoreType.DMA,
        ],
    )


_BLK = 2000


def _quant_body(part_ref, gama_ref, bit_ref, fea_ref,
                out_ref, bs_ref, sq_scr):
    i = pl.program_id(0)

    @pl.when(i == 0)
    def _():
        p = part_ref[...]
        deg = p[0] + p[1]
        idx = jnp.minimum(deg, _ND - 1)
        kk = lax.broadcasted_iota(jnp.int32, (_ND, _N), 0)
        onehot_t = (kk == idx).astype(jnp.float32)

        g = jnp.maximum(jnp.abs(gama_ref[...]), 1e-12)
        b = jnp.clip(jnp.round(bit_ref[...]), 1.0, 32.0)
        qm = jnp.exp2(b - 1.0) - 1.0
        tab = jnp.concatenate([g, qm], axis=1)
        sq_scr[...] = lax.dot_general(
            onehot_t, tab,
            dimension_numbers=(((0,), (0,)), ((), ())),
            preferred_element_type=jnp.float32)

        occ = jnp.max(onehot_t, axis=1, keepdims=True)
        total = lax.dot_general(occ, bit_ref[...],
                                dimension_numbers=(((0,), (0,)), ((), ())),
                                preferred_element_type=jnp.float32)
        bs_ref[...] = total * (_D / 8.0 / 1024.0)

    sq = sq_scr[pl.ds(i * _BLK, _BLK), :]
    sc = sq[:, 0:1]
    qmax = sq[:, 1:2]
    q = jnp.clip(jnp.round(fea_ref[...] / sc), -qmax, qmax)
    out_ref[...] = q * sc


def _quant_tc(partials, gama, bit, fea):
    return pl.pallas_call(
        _quant_body,
        grid=(_N // _BLK,),
        in_specs=[
            pl.BlockSpec((2, 1, _N), lambda i: (0, 0, 0)),
            pl.BlockSpec((_ND, 1), lambda i: (0, 0)),
            pl.BlockSpec((_ND, 1), lambda i: (0, 0)),
            pl.BlockSpec((_BLK, _D), lambda i: (i, 0)),
        ],
        out_specs=[
            pl.BlockSpec((_BLK, _D), lambda i: (i, 0)),
            pl.BlockSpec((1, 1), lambda i: (0, 0)),
        ],
        out_shape=[
            jax.ShapeDtypeStruct((_N, _D), jnp.float32),
            jax.ShapeDtypeStruct((1, 1), jnp.float32),
        ],
        scratch_shapes=[pltpu.VMEM((_N, 2), jnp.float32)],
    )(partials, gama, bit, fea)


def kernel(fea, edge_index, gama, bit):
    partials = _deg_sc()(edge_index)
    fea_q, bs = _quant_tc(partials, gama, bit, fea)
    return fea_q, bs[0, 0]


# speedup vs baseline: 1.0391x; 1.0055x over previous
import functools

import jax
import jax.numpy as jnp
from jax import lax
from jax.experimental import pallas as pl
from jax.experimental.pallas import tpu as pltpu
from jax.experimental.pallas import tpu_sc as plsc

_N = 10000
_E = 320000
_D = 128
_ND = 128

_BATCH = 128
_ROWS_PER_TILE = 100
_NTILES = 25


def _deg_body(edges_hbm, out_hbm, idx2_v, ones_v, zeros_v, hist_s, sem):
    c = lax.axis_index("c")
    s = lax.axis_index("s")
    wid = s * 2 + c

    widc = jnp.minimum(wid, _NTILES - 1)
    src = edges_hbm.at[:, pl.ds(widc * _ROWS_PER_TILE * _BATCH,
                                _ROWS_PER_TILE * _BATCH)]

    @pl.when(wid < _NTILES)
    def _():
        pltpu.async_copy(src, idx2_v, sem)

    zero16 = jnp.zeros((16,), jnp.int32)
    one16 = jnp.ones((16,), jnp.int32)

    def fill(j, carry):
        zeros_v[pl.ds(j * 16, 16)] = zero16
        return carry

    lax.fori_loop(0, 40, fill, 0)

    def fill1(j, carry):
        ones_v[pl.ds(j * 16, 16)] = one16
        return carry

    lax.fori_loop(0, 8, fill1, 0)

    @pl.when(s < 15)
    def _():
        pltpu.sync_copy(zeros_v, hist_s.at[pl.ds(s * 640, 640)])

    @pl.when(s == 15)
    def _():
        pltpu.sync_copy(zeros_v.at[pl.ds(0, 400)], hist_s.at[pl.ds(9600, 400)])

    @pl.when(wid < _NTILES)
    def _():
        pltpu.make_async_copy(src, idx2_v, sem).wait()

    plsc.subcore_barrier()

    @pl.when(wid < _NTILES)
    def _():
        def fire(base):
            for u in range(5):
                pltpu.async_copy(
                    ones_v,
                    hist_s.at[idx2_v.at[1, pl.ds((base + u) * _BATCH,
                                                 _BATCH)]],
                    sem, add=True)

        def drain5():
            for _u in range(5):
                pltpu.make_async_copy(
                    ones_v, hist_s.at[idx2_v.at[1, pl.ds(0, _BATCH)]],
                    sem).wait()

        fire(0)
        fire(5)

        def chunk(cj, carry):
            fire(cj * 5)
            drain5()
            return carry

        lax.fori_loop(2, _ROWS_PER_TILE // 5, chunk, 0)
        drain5()
        drain5()

    plsc.subcore_barrier()

    @pl.when(s == 0)
    def _():
        pltpu.sync_copy(hist_s, out_hbm.at[c, 0])


@functools.cache
def _deg_sc():
    return pl.kernel(
        _deg_body,
        mesh=plsc.VectorSubcoreMesh(core_axis_name="c", subcore_axis_name="s"),
        out_type=jax.ShapeDtypeStruct((2, 1, _N), jnp.int32),
        scratch_types=[
            pltpu.VMEM((2, _ROWS_PER_TILE * _BATCH), jnp.int32),
            pltpu.VMEM((_BATCH,), jnp.int32),
            pltpu.VMEM((640,), jnp.int32),
            pltpu.VMEM_SHARED((_N,), jnp.int32),
            pltpu.SemaphoreType.DMA,
        ],
    )


_BLK = 2000


def _quant_body(part_ref, gama_ref, bit_ref, fea_ref,
                out_ref, bs_ref, sq_scr):
    i = pl.program_id(0)

    @pl.when(i == 0)
    def _():
        p = part_ref[...]
        deg = p[0] + p[1]
        idx = jnp.minimum(deg, _ND - 1)
        kk = lax.broadcasted_iota(jnp.int32, (_ND, _N), 0)
        onehot_t = (kk == idx).astype(jnp.float32)

        g = jnp.maximum(jnp.abs(gama_ref[...]), 1e-12)
        b = jnp.clip(jnp.round(bit_ref[...]), 1.0, 32.0)
        qm = jnp.exp2(b - 1.0) - 1.0
        tab = jnp.concatenate([g, qm], axis=1)
        sq_scr[...] = lax.dot_general(
            onehot_t, tab,
            dimension_numbers=(((0,), (0,)), ((), ())),
            preferred_element_type=jnp.float32)

        occ = jnp.max(onehot_t, axis=1, keepdims=True)
        total = lax.dot_general(occ, bit_ref[...],
                                dimension_numbers=(((0,), (0,)), ((), ())),
                                preferred_element_type=jnp.float32)
        bs_ref[...] = total * (_D / 8.0 / 1024.0)

    sq = sq_scr[pl.ds(i * _BLK, _BLK), :]
    sc = sq[:, 0:1]
    qmax = sq[:, 1:2]
    q = jnp.clip(jnp.round(fea_ref[...] / sc), -qmax, qmax)
    out_ref[...] = q * sc


def _quant_tc(partials, gama, bit, fea):
    return pl.pallas_call(
        _quant_body,
        grid=(_N // _BLK,),
        in_specs=[
            pl.BlockSpec((2, 1, _N), lambda i: (0, 0, 0)),
            pl.BlockSpec((_ND, 1), lambda i: (0, 0)),
            pl.BlockSpec((_ND, 1), lambda i: (0, 0)),
            pl.BlockSpec((_BLK, _D), lambda i: (i, 0)),
        ],
        out_specs=[
            pl.BlockSpec((_BLK, _D), lambda i: (i, 0)),
            pl.BlockSpec((1, 1), lambda i: (0, 0)),
        ],
        out_shape=[
            jax.ShapeDtypeStruct((_N, _D), jnp.float32),
            jax.ShapeDtypeStruct((1, 1), jnp.float32),
        ],
        scratch_shapes=[pltpu.VMEM((_N, 2), jnp.float32)],
    )(partials, gama, bit, fea)


def kernel(fea, edge_index, gama, bit):
    partials = _deg_sc()(edge_index)
    fea_q, bs = _quant_tc(partials, gama, bit, fea)
    return fea_q, bs[0, 0]
